# baseline (speedup 1.0000x reference)
import functools

import jax
import jax.numpy as jnp
from jax.experimental import pallas as pl
from jax.experimental.pallas import tpu as pltpu

_ALPHA = 200.0
_BLK = 128
_NROW = 2048
_NCOL = 2048


def _voting_kernel(nrow_ref, s_ref, o_ref):
    b = pl.program_id(0)
    i = pl.program_id(1)
    n = nrow_ref[b]
    row0 = i * _BLK

    @pl.when(row0 >= n)
    def _():
        o_ref[...] = jnp.zeros_like(o_ref)

    @pl.when(row0 < n)
    def _():
        x = s_ref[0] * _ALPHA
        m = jnp.max(x, axis=-1, keepdims=True)
        e = jnp.exp(x - m)
        sm = e / jnp.sum(e, axis=-1, keepdims=True)
        row = row0 + jax.lax.broadcasted_iota(jnp.int32, (_BLK, _NCOL), 0)
        o_ref[0] = jnp.where(row < n, sm, 0.0)


def _s_index_map(b, i, nrow_ref):
    n = nrow_ref[b]
    last_valid = jnp.maximum(pl.cdiv(n, _BLK) - 1, 0)
    return b, jnp.minimum(i, last_valid), 0


@jax.jit
def kernel(s, nrow_gt):
    grid_spec = pltpu.PrefetchScalarGridSpec(
        num_scalar_prefetch=1,
        grid=(s.shape[0], _NROW // _BLK),
        in_specs=[
            pl.BlockSpec((1, _BLK, _NCOL), _s_index_map),
        ],
        out_specs=pl.BlockSpec((1, _BLK, _NCOL), lambda b, i, n_ref: (b, i, 0)),
    )
    return pl.pallas_call(
        _voting_kernel,
        grid_spec=grid_spec,
        out_shape=jax.ShapeDtypeStruct(s.shape, s.dtype),
    )(nrow_gt, s)


# speedup vs baseline: 1.4732x; 1.4732x over previous
import functools

import jax
import jax.numpy as jnp
from jax.experimental import pallas as pl
from jax.experimental.pallas import tpu as pltpu

_ALPHA = 200.0
_BLK = 512
_NROW = 2048
_NCOL = 2048


def _voting_kernel(nrow_ref, s_ref, o_ref):
    b = pl.program_id(0)
    i = pl.program_id(1)
    n = nrow_ref[b]
    row0 = i * _BLK

    @pl.when(row0 >= n)
    def _():
        o_ref[...] = jnp.zeros_like(o_ref)

    @pl.when(row0 < n)
    def _():
        x = s_ref[0] * _ALPHA
        m = jnp.max(x, axis=-1, keepdims=True)
        e = jnp.exp(x - m)
        sm = e / jnp.sum(e, axis=-1, keepdims=True)
        row = row0 + jax.lax.broadcasted_iota(jnp.int32, (_BLK, _NCOL), 0)
        o_ref[0] = jnp.where(row < n, sm, 0.0)


def _s_index_map(b, i, nrow_ref):
    n = nrow_ref[b]
    last_valid = jnp.maximum(pl.cdiv(n, _BLK) - 1, 0)
    return b, jnp.minimum(i, last_valid), 0


@jax.jit
def kernel(s, nrow_gt):
    grid_spec = pltpu.PrefetchScalarGridSpec(
        num_scalar_prefetch=1,
        grid=(s.shape[0], _NROW // _BLK),
        in_specs=[
            pl.BlockSpec((1, _BLK, _NCOL), _s_index_map),
        ],
        out_specs=pl.BlockSpec((1, _BLK, _NCOL), lambda b, i, n_ref: (b, i, 0)),
    )
    return pl.pallas_call(
        _voting_kernel,
        grid_spec=grid_spec,
        out_shape=jax.ShapeDtypeStruct(s.shape, s.dtype),
    )(nrow_gt, s)
